# Initial kernel scaffold; baseline (speedup 1.0000x reference)
import jax
import jax.numpy as jnp
from jax.experimental import pallas as pl


def kernel(x, edge_index, W, att_src, att_dst, bias, gamma, beta):
    raise NotImplementedError("write your pallas kernel here")


# speedup vs baseline: 13.3646x; 13.3646x over previous
import jax
import jax.numpy as jnp
from jax import lax
from jax.experimental import pallas as pl
from jax.experimental.pallas import tpu as pltpu
from jax.experimental.pallas import tpu_sc as plsc

N = 10000
E = 320000
D = 128
H = 4
C = 32
NC = 2
NS = 16
HPC = H // NC
DH = D // NC
NACC = 10240
RPT = NACC // NS
EPW = E // NS
K = 80
NCHUNK = EPW // K
DROWS = NACC // 16
DRPT = DROWS // NS
ROWBLK = 1000


def _pre_body(x_ref, w_ref, acat_ref, h_ref, sd_ref):
    h = jnp.dot(x_ref[...], w_ref[...], preferred_element_type=jnp.float32)
    h_ref[...] = h
    sd_ref[...] = jnp.dot(h, acat_ref[...], preferred_element_type=jnp.float32)


def _pre(x, W, acat):
    return pl.pallas_call(
        _pre_body,
        grid=(N // ROWBLK,),
        in_specs=[
            pl.BlockSpec((ROWBLK, D), lambda i: (i, 0)),
            pl.BlockSpec((D, D), lambda i: (0, 0)),
            pl.BlockSpec((D, 2 * H), lambda i: (0, 0)),
        ],
        out_specs=[
            pl.BlockSpec((ROWBLK, D), lambda i: (i, 0)),
            pl.BlockSpec((ROWBLK, 2 * H), lambda i: (i, 0)),
        ],
        out_shape=[
            jax.ShapeDtypeStruct((N, D), jnp.float32),
            jax.ShapeDtypeStruct((N, 2 * H), jnp.float32),
        ],
    )(x, W, acat)


def _sc_body(hsplit_hbm, asrc_hbm, adst_hbm, src_hbm, dst_hbm,
             num_hbm, den_hbm,
             asrc_v, adst_v, src_v, dst_v, src2_v, idx_v, rows_v, den_t,
             zbuf, num_sp, den_sp, gsem):
    cid = lax.axis_index("c")
    sid = lax.axis_index("s")
    iota = lax.iota(jnp.int32, 16)
    zeros16 = jnp.zeros((16,), jnp.float32)

    pltpu.sync_copy(asrc_hbm.at[cid], asrc_v)
    pltpu.sync_copy(adst_hbm.at[cid], adst_v)

    @pl.loop(0, K * DH // 16)
    def _zz(r):
        zbuf[r // (DH // 16), pl.ds((r % (DH // 16)) * 16, 16)] = zeros16

    @pl.loop(0, DROWS * 32 // 16)
    def _zd(r):
        den_t[r // 2, pl.ds((r % 2) * 16, 16)] = zeros16

    @pl.loop(0, DROWS // 16)
    def _zi(g):
        idx_v[g // 8, pl.ds((g % 8) * 16, 16)] = iota + g * 16

    @pl.loop(0, RPT // K)
    def _zacc(j):
        pltpu.sync_copy(zbuf, num_sp.at[pl.ds(sid * RPT + j * K, K)])
    pltpu.sync_copy(den_t.at[pl.ds(0, DRPT)],
                    den_sp.at[pl.ds(sid * DRPT, DRPT)])
    plsc.subcore_barrier()

    @pl.loop(0, NCHUNK)
    def _chunk(i):
        base = sid * EPW + i * K
        pltpu.sync_copy(src_hbm.at[pl.ds(base, K)], src_v)
        pltpu.sync_copy(dst_hbm.at[pl.ds(base, K)], dst_v)
        for g in range(K // 16):
            sl = pl.ds(g * 16, 16)
            src2_v[sl] = src_v[sl] * NC + cid
        pltpu.async_copy(hsplit_hbm.at[src2_v], rows_v, gsem).wait()

        @pl.loop(0, K // 16)
        def _group(g):
            kvec = iota + (g * 16)
            srcv = src_v[pl.ds(g * 16, 16)]
            dstv = dst_v[pl.ds(g * 16, 16)]
            for hh in range(HPC):
                sv = plsc.load_gather(asrc_v, [srcv * HPC + hh])
                dv = plsc.load_gather(adst_v, [dstv * HPC + hh])
                al = sv + dv
                al = jnp.where(al >= 0.0, al, al * 0.2)
                exv = jnp.exp(al)
                plsc.addupdate_scatter(
                    den_t,
                    [lax.shift_right_logical(dstv, 4),
                     lax.shift_left(dstv & 15, 1) + hh],
                    exv)

                @pl.loop(0, C, unroll=4)
                def _col(cc):
                    colv = jnp.full((16,), hh * C, jnp.int32) + cc
                    v = plsc.load_gather(rows_v, [kvec, colv])
                    plsc.store_scatter(rows_v, [kvec, colv], v * exv)

        pltpu.sync_copy(rows_v, num_sp.at[dst_v], add=True)

    for b in range(DROWS // 128):
        pltpu.sync_copy(den_t.at[pl.ds(b * 128, 128)],
                        den_sp.at[idx_v.at[b]], add=True)
    plsc.subcore_barrier()
    pltpu.sync_copy(num_sp.at[pl.ds(sid * RPT, RPT)],
                    num_hbm.at[cid, pl.ds(sid * RPT, RPT)])
    pltpu.sync_copy(den_sp.at[pl.ds(sid * DRPT, DRPT)],
                    den_hbm.at[cid, pl.ds(sid * DRPT, DRPT)])


def _sc(hsplit, asrc2, adst2, src, dst):
    mesh = plsc.VectorSubcoreMesh(core_axis_name="c", subcore_axis_name="s",
                                  num_cores=NC, num_subcores=NS)
    f = pl.kernel(
        _sc_body,
        out_type=[
            jax.ShapeDtypeStruct((NC, NACC, DH), jnp.float32),
            jax.ShapeDtypeStruct((NC, DROWS, 32), jnp.float32),
        ],
        mesh=mesh,
        compiler_params=pltpu.CompilerParams(use_tc_tiling_on_sc=False,
                                             needs_layout_passes=False),
        scratch_types=[
            pltpu.VMEM((N * HPC,), jnp.float32),
            pltpu.VMEM((N * HPC,), jnp.float32),
            pltpu.VMEM((K,), jnp.int32),
            pltpu.VMEM((K,), jnp.int32),
            pltpu.VMEM((K,), jnp.int32),
            pltpu.VMEM((DROWS // 128, 128), jnp.int32),
            pltpu.VMEM((K, DH), jnp.float32),
            pltpu.VMEM((DROWS, 32), jnp.float32),
            pltpu.VMEM((K, DH), jnp.float32),
            pltpu.VMEM_SHARED((NACC, DH), jnp.float32),
            pltpu.VMEM_SHARED((DROWS, 32), jnp.float32),
            pltpu.SemaphoreType.DMA,
        ],
    )
    return f(hsplit, asrc2, adst2, src, dst)


def _post_body(part_ref, den4_ref, h_ref, sd_ref, eh_ref, bias_ref,
               gamma_ref, beta_ref, out_ref):
    sd = sd_ref[...]
    a = sd[:, :H] + sd[:, H:]
    ex_self = jnp.exp(jnp.where(a >= 0.0, a, a * 0.2))
    eh = eh_ref[...]
    p0 = part_ref[0]
    p1 = part_ref[1]
    hm = h_ref[...]
    num = jnp.concatenate([p0, p1], axis=-1)
    num = num + jnp.dot(ex_self, eh, preferred_element_type=jnp.float32) * hm
    den4 = den4_ref[...] + ex_self
    den = jnp.dot(den4, eh, preferred_element_type=jnp.float32)
    y = num / den + bias_ref[...]
    mu = jnp.mean(y, axis=-1, keepdims=True)
    d = y - mu
    var = jnp.mean(d * d, axis=-1, keepdims=True)
    out_ref[...] = d * lax.rsqrt(var + 1e-5) * gamma_ref[...] + beta_ref[...]


def _post(part, den4, h, sd, eh, bias, gamma, beta):
    return pl.pallas_call(
        _post_body,
        grid=(N // ROWBLK,),
        in_specs=[
            pl.BlockSpec((NC, ROWBLK, DH), lambda i: (0, i, 0)),
            pl.BlockSpec((ROWBLK, H), lambda i: (i, 0)),
            pl.BlockSpec((ROWBLK, D), lambda i: (i, 0)),
            pl.BlockSpec((ROWBLK, 2 * H), lambda i: (i, 0)),
            pl.BlockSpec((H, D), lambda i: (0, 0)),
            pl.BlockSpec((1, D), lambda i: (0, 0)),
            pl.BlockSpec((1, D), lambda i: (0, 0)),
            pl.BlockSpec((1, D), lambda i: (0, 0)),
        ],
        out_specs=pl.BlockSpec((ROWBLK, D), lambda i: (i, 0)),
        out_shape=jax.ShapeDtypeStruct((N, D), jnp.float32),
    )(part, den4, h, sd, eh, bias, gamma, beta)


def kernel(x, edge_index, W, att_src, att_dst, bias, gamma, beta):
    src = edge_index[0].astype(jnp.int32)
    dst = edge_index[1].astype(jnp.int32)
    m = (jnp.arange(D, dtype=jnp.int32)[:, None] // C
         == jnp.arange(H, dtype=jnp.int32)[None, :]).astype(jnp.float32)
    acat = jnp.concatenate(
        [m * att_src.reshape(D)[:, None], m * att_dst.reshape(D)[:, None]],
        axis=1)
    eh = m.T

    h, sd = _pre(x, W, acat)
    hsplit = h.reshape(N * NC, DH)
    asrc2 = sd[:, :H].reshape(N, NC, HPC).transpose(1, 0, 2).reshape(NC, N * HPC)
    adst2 = sd[:, H:].reshape(N, NC, HPC).transpose(1, 0, 2).reshape(NC, N * HPC)
    num, den = _sc(hsplit, asrc2, adst2, src, dst)
    den4 = jnp.concatenate(
        [den[0].reshape(DROWS * 16, HPC), den[1].reshape(DROWS * 16, HPC)],
        axis=-1)[:N]
    return _post(num, den4, h, sd, eh, bias.reshape(1, D), gamma.reshape(1, D),
                 beta.reshape(1, D))


# speedup vs baseline: 14.6764x; 1.0982x over previous
import jax
import jax.numpy as jnp
from jax import lax
from jax.experimental import pallas as pl
from jax.experimental.pallas import tpu as pltpu
from jax.experimental.pallas import tpu_sc as plsc

N = 10000
E = 320000
D = 128
H = 4
C = 32
NC = 2
NS = 16
HPC = H // NC
DH = D // NC
NACC = 10240
RPT = NACC // NS
EPW = E // NS
K = 80
NCHUNK = EPW // K
DROWS = NACC // 16
DRPT = DROWS // NS
ROWBLK = 1000


def _pre_body(x_ref, w_ref, acat_ref, h_ref, sd_ref):
    h = jnp.dot(x_ref[...], w_ref[...], preferred_element_type=jnp.float32)
    h_ref[...] = h
    sd_ref[...] = jnp.dot(h, acat_ref[...], preferred_element_type=jnp.float32)


def _pre(x, W, acat):
    return pl.pallas_call(
        _pre_body,
        grid=(N // ROWBLK,),
        in_specs=[
            pl.BlockSpec((ROWBLK, D), lambda i: (i, 0)),
            pl.BlockSpec((D, D), lambda i: (0, 0)),
            pl.BlockSpec((D, 2 * H), lambda i: (0, 0)),
        ],
        out_specs=[
            pl.BlockSpec((ROWBLK, D), lambda i: (i, 0)),
            pl.BlockSpec((ROWBLK, 2 * H), lambda i: (i, 0)),
        ],
        out_shape=[
            jax.ShapeDtypeStruct((N, D), jnp.float32),
            jax.ShapeDtypeStruct((N, 2 * H), jnp.float32),
        ],
    )(x, W, acat)


def _sc_body(hsplit_hbm, asrc_hbm, adst_hbm, src_hbm, dst_hbm,
             num_hbm, den_hbm,
             asrc_v, adst_v, srcc_v, src2_v, dst2_v, idx_v, rows_v,
             den_t, num_sp, den_sp, gsem0, gsem1):
    cid = lax.axis_index("c")
    sid = lax.axis_index("s")
    iota = lax.iota(jnp.int32, 16)
    zeros16 = jnp.zeros((16,), jnp.float32)
    gsems = (gsem0, gsem1)

    pltpu.sync_copy(asrc_hbm.at[cid], asrc_v)
    pltpu.sync_copy(adst_hbm.at[cid], adst_v)

    @pl.loop(0, K * DH // 16)
    def _zz(r):
        rows_v[0, r // (DH // 16), pl.ds((r % (DH // 16)) * 16, 16)] = zeros16

    @pl.loop(0, DROWS * 32 // 16)
    def _zd(r):
        den_t[r // 2, pl.ds((r % 2) * 16, 16)] = zeros16

    @pl.loop(0, DROWS // 16)
    def _zi(g):
        idx_v[g // 8, pl.ds((g % 8) * 16, 16)] = iota + g * 16

    @pl.loop(0, RPT // K)
    def _zacc(j):
        pltpu.sync_copy(rows_v.at[0], num_sp.at[pl.ds(sid * RPT + j * K, K)])
    pltpu.sync_copy(den_t.at[pl.ds(0, DRPT)],
                    den_sp.at[pl.ds(sid * DRPT, DRPT)])

    def _stage(c, b):
        off = sid * EPW + c * K
        pltpu.sync_copy(src_hbm.at[pl.ds(off, K)], srcc_v.at[b])
        pltpu.sync_copy(dst_hbm.at[pl.ds(off, K)], dst2_v.at[b])
        for g in range(K // 16):
            sl = pl.ds(g * 16, 16)
            src2_v[b, sl] = srcc_v[b, sl] * NC + cid
        pltpu.make_async_copy(hsplit_hbm.at[src2_v.at[b]], rows_v.at[b],
                              gsems[b]).start()

    _stage(0, 0)
    _stage(1, 1)
    plsc.subcore_barrier()

    @pl.loop(0, NCHUNK, step=2)
    def _chunk(i):
        for b in range(2):
            c = i + b
            pltpu.make_async_copy(hsplit_hbm.at[src2_v.at[b]], rows_v.at[b],
                                  gsems[b]).wait()

            @pl.loop(0, K // 16)
            def _group(g):
                kvec = iota + (g * 16)
                srcv = srcc_v[b, pl.ds(g * 16, 16)]
                dstv = dst2_v[b, pl.ds(g * 16, 16)]
                for hh in range(HPC):
                    sv = plsc.load_gather(asrc_v, [srcv * HPC + hh])
                    dv = plsc.load_gather(adst_v, [dstv * HPC + hh])
                    al = sv + dv
                    al = jnp.where(al >= 0.0, al, al * 0.2)
                    exv = jnp.exp(al)
                    plsc.addupdate_scatter(
                        den_t,
                        [lax.shift_right_logical(dstv, 4),
                         lax.shift_left(dstv & 15, 1) + hh],
                        exv)

                    @pl.loop(0, C, unroll=4)
                    def _col(cc):
                        colv = jnp.full((16,), hh * C, jnp.int32) + cc
                        v = plsc.load_gather(rows_v.at[b], [kvec, colv])
                        plsc.store_scatter(rows_v.at[b], [kvec, colv],
                                           v * exv)

            pltpu.sync_copy(rows_v.at[b], num_sp.at[dst2_v.at[b]], add=True)

            @pl.when(c + 2 < NCHUNK)
            def _pf():
                _stage(c + 2, b)

    for bb in range(DROWS // 128):
        pltpu.sync_copy(den_t.at[pl.ds(bb * 128, 128)],
                        den_sp.at[idx_v.at[bb]], add=True)
    plsc.subcore_barrier()
    pltpu.sync_copy(num_sp.at[pl.ds(sid * RPT, RPT)],
                    num_hbm.at[cid, pl.ds(sid * RPT, RPT)])
    pltpu.sync_copy(den_sp.at[pl.ds(sid * DRPT, DRPT)],
                    den_hbm.at[cid, pl.ds(sid * DRPT, DRPT)])


def _sc(hsplit, asrc2, adst2, src, dst):
    mesh = plsc.VectorSubcoreMesh(core_axis_name="c", subcore_axis_name="s",
                                  num_cores=NC, num_subcores=NS)
    f = pl.kernel(
        _sc_body,
        out_type=[
            jax.ShapeDtypeStruct((NC, NACC, DH), jnp.float32),
            jax.ShapeDtypeStruct((NC, DROWS, 32), jnp.float32),
        ],
        mesh=mesh,
        compiler_params=pltpu.CompilerParams(use_tc_tiling_on_sc=False,
                                             needs_layout_passes=False),
        scratch_types=[
            pltpu.VMEM((N * HPC,), jnp.float32),
            pltpu.VMEM((N * HPC,), jnp.float32),
            pltpu.VMEM((2, K), jnp.int32),
            pltpu.VMEM((2, K), jnp.int32),
            pltpu.VMEM((2, K), jnp.int32),
            pltpu.VMEM((DROWS // 128, 128), jnp.int32),
            pltpu.VMEM((2, K, DH), jnp.float32),
            pltpu.VMEM((DROWS, 32), jnp.float32),
            pltpu.VMEM_SHARED((NACC, DH), jnp.float32),
            pltpu.VMEM_SHARED((DROWS, 32), jnp.float32),
            pltpu.SemaphoreType.DMA,
            pltpu.SemaphoreType.DMA,
        ],
    )
    return f(hsplit, asrc2, adst2, src, dst)


def _post_body(part_ref, den4_ref, h_ref, sd_ref, eh_ref, bias_ref,
               gamma_ref, beta_ref, out_ref):
    sd = sd_ref[...]
    a = sd[:, :H] + sd[:, H:]
    ex_self = jnp.exp(jnp.where(a >= 0.0, a, a * 0.2))
    eh = eh_ref[...]
    p0 = part_ref[0]
    p1 = part_ref[1]
    hm = h_ref[...]
    num = jnp.concatenate([p0, p1], axis=-1)
    num = num + jnp.dot(ex_self, eh, preferred_element_type=jnp.float32) * hm
    den4 = den4_ref[...] + ex_self
    den = jnp.dot(den4, eh, preferred_element_type=jnp.float32)
    y = num / den + bias_ref[...]
    mu = jnp.mean(y, axis=-1, keepdims=True)
    d = y - mu
    var = jnp.mean(d * d, axis=-1, keepdims=True)
    out_ref[...] = d * lax.rsqrt(var + 1e-5) * gamma_ref[...] + beta_ref[...]


def _post(part, den4, h, sd, eh, bias, gamma, beta):
    return pl.pallas_call(
        _post_body,
        grid=(N // ROWBLK,),
        in_specs=[
            pl.BlockSpec((NC, ROWBLK, DH), lambda i: (0, i, 0)),
            pl.BlockSpec((ROWBLK, H), lambda i: (i, 0)),
            pl.BlockSpec((ROWBLK, D), lambda i: (i, 0)),
            pl.BlockSpec((ROWBLK, 2 * H), lambda i: (i, 0)),
            pl.BlockSpec((H, D), lambda i: (0, 0)),
            pl.BlockSpec((1, D), lambda i: (0, 0)),
            pl.BlockSpec((1, D), lambda i: (0, 0)),
            pl.BlockSpec((1, D), lambda i: (0, 0)),
        ],
        out_specs=pl.BlockSpec((ROWBLK, D), lambda i: (i, 0)),
        out_shape=jax.ShapeDtypeStruct((N, D), jnp.float32),
    )(part, den4, h, sd, eh, bias, gamma, beta)


def kernel(x, edge_index, W, att_src, att_dst, bias, gamma, beta):
    src = edge_index[0].astype(jnp.int32)
    dst = edge_index[1].astype(jnp.int32)
    m = (jnp.arange(D, dtype=jnp.int32)[:, None] // C
         == jnp.arange(H, dtype=jnp.int32)[None, :]).astype(jnp.float32)
    acat = jnp.concatenate(
        [m * att_src.reshape(D)[:, None], m * att_dst.reshape(D)[:, None]],
        axis=1)
    eh = m.T

    h, sd = _pre(x, W, acat)
    hsplit = h.reshape(N * NC, DH)
    asrc2 = sd[:, :H].reshape(N, NC, HPC).transpose(1, 0, 2).reshape(NC, N * HPC)
    adst2 = sd[:, H:].reshape(N, NC, HPC).transpose(1, 0, 2).reshape(NC, N * HPC)
    num, den = _sc(hsplit, asrc2, adst2, src, dst)
    den4 = jnp.concatenate(
        [den[0].reshape(DROWS * 16, HPC), den[1].reshape(DROWS * 16, HPC)],
        axis=-1)[:N]
    return _post(num, den4, h, sd, eh, bias.reshape(1, D), gamma.reshape(1, D),
                 beta.reshape(1, D))
